# baseline (speedup 1.0000x reference)
import jax
import jax.numpy as jnp
from jax.experimental import pallas as pl
from jax.experimental.pallas import tpu as pltpu

_B, _H, _SEQ, _D = 8, 8, 4096, 128
_S = 16
_NCH = 16
_CH = _SEQ // _NCH


def _body(off_ref, xk, xv, kin, vin, kout, vout, semk, semv, semu):
    off = off_ref[0]
    cstar = off // _CH

    def _chunk(c):
        sl = pl.ds(c * _CH, _CH)
        return (
            pltpu.make_async_copy(kin.at[:, :, sl, :], kout.at[:, :, sl, :],
                                  semk.at[c]),
            pltpu.make_async_copy(vin.at[:, :, sl, :], vout.at[:, :, sl, :],
                                  semv.at[c]),
        )

    for c in range(_NCH):
        ck, cv = _chunk(c)
        ck.start()
        cv.start()

    slstar = pl.ds(cstar * _CH, _CH)
    pltpu.make_async_copy(kin.at[:, :, slstar, :], kout.at[:, :, slstar, :],
                          semk.at[cstar]).wait()
    pltpu.make_async_copy(vin.at[:, :, slstar, :], vout.at[:, :, slstar, :],
                          semv.at[cstar]).wait()

    for h in range(_H):
        for b in range(_B):
            pltpu.make_async_copy(
                xk.at[:, h, b, :], kout.at[b, h, pl.ds(off, _S), :], semu
            ).start()
            pltpu.make_async_copy(
                xv.at[:, h, b, :], vout.at[b, h, pl.ds(off, _S), :], semu
            ).start()

    for c in range(_NCH):
        ck, cv = _chunk(c)

        @pl.when(c != cstar)
        def _wait_bulk(ck=ck, cv=cv):
            ck.wait()
            cv.wait()

    slu = pl.ds(off, _S)
    pltpu.make_async_copy(kin.at[:, :, slu, :], kout.at[:, :, slu, :],
                          semu).wait()
    pltpu.make_async_copy(vin.at[:, :, slu, :], vout.at[:, :, slu, :],
                          semu).wait()


def kernel(xk, xv, key_past, value_past, layer_past_len):
    off = jnp.asarray(layer_past_len, jnp.int32).reshape((1,))
    out_sd = jax.ShapeDtypeStruct((_B, _H, _SEQ, _D), key_past.dtype)
    any_spec = pl.BlockSpec(memory_space=pl.ANY)
    new_key, new_value = pl.pallas_call(
        _body,
        in_specs=[
            pl.BlockSpec(memory_space=pltpu.SMEM),
            any_spec, any_spec, any_spec, any_spec,
        ],
        out_specs=[any_spec, any_spec],
        out_shape=[out_sd, out_sd],
        scratch_shapes=[
            pltpu.SemaphoreType.DMA((_NCH,)),
            pltpu.SemaphoreType.DMA((_NCH,)),
            pltpu.SemaphoreType.DMA,
        ],
    )(off, xk, xv, key_past, value_past)
    return (new_key, new_value)


# speedup vs baseline: 48.6742x; 48.6742x over previous
import jax
import jax.numpy as jnp
from jax.experimental import pallas as pl
from jax.experimental.pallas import tpu as pltpu

_B, _H, _SEQ, _D = 8, 8, 4096, 128
_S = 16
_CHUNK = 1024


def _body(off_ref, xk_ref, xv_ref, kin_ref, vin_ref, kout_ref, vout_ref):
    h = pl.program_id(0)
    j = pl.program_id(1)
    kout_ref[...] = kin_ref[...]
    vout_ref[...] = vin_ref[...]
    off = off_ref[0]

    @pl.when(j == off // _CHUNK)
    def _update():
        local = off - j * _CHUNK
        for s in range(_S):
            kout_ref[:, 0, pl.ds(local + s, 1), :] = (
                xk_ref[s, h, :, :].reshape(_B, 1, _D))
            vout_ref[:, 0, pl.ds(local + s, 1), :] = (
                xv_ref[s, h, :, :].reshape(_B, 1, _D))


def kernel(xk, xv, key_past, value_past, layer_past_len):
    off = jnp.asarray(layer_past_len, jnp.int32).reshape((1,))
    out_sd = jax.ShapeDtypeStruct((_B, _H, _SEQ, _D), key_past.dtype)
    grid = (_H, _SEQ // _CHUNK)
    cache_spec = pl.BlockSpec(
        (_B, 1, _CHUNK, _D), lambda h, j: (0, h, j, 0))
    x_spec = pl.BlockSpec((_S, _H, _B, _D), lambda h, j: (0, 0, 0, 0))
    new_key, new_value = pl.pallas_call(
        _body,
        grid=grid,
        in_specs=[
            pl.BlockSpec(memory_space=pltpu.SMEM),
            x_spec,
            x_spec,
            cache_spec,
            cache_spec,
        ],
        out_specs=[cache_spec, cache_spec],
        out_shape=[out_sd, out_sd],
    )(off, xk, xv, key_past, value_past)
    return (new_key, new_value)
